# baseline (speedup 1.0000x reference)
import functools

import jax
import jax.numpy as jnp
from jax import lax
from jax.experimental import pallas as pl
from jax.experimental.pallas import tpu as pltpu
from jax.experimental.pallas import tpu_sc as plsc

N = 10000
E = 320000
D = 128

NC = 2
NS = 16
NW = NC * NS
CH = 128
NCH = 80
EPW = NCH * CH
E_PAD = NW * EPW
ACC_ROWS = 10240
RPS = ACC_ROWS // NS
ZR = 64
DEG_W = 16

BLK = 2000


_MESH = functools.partial(
    plsc.VectorSubcoreMesh, core_axis_name="c", subcore_axis_name="s"
)


def _sc_deg(dst3):
    @functools.partial(
        pl.kernel,
        out_type=jax.ShapeDtypeStruct((NC, N, DEG_W), jnp.float32),
        mesh=_MESH(),
        scratch_types=[
            pltpu.VMEM((NCH, CH), jnp.int32),
            pltpu.VMEM((CH, DEG_W), jnp.float32),
            pltpu.VMEM((ZR, DEG_W), jnp.float32),
            pltpu.VMEM_SHARED((ACC_ROWS, DEG_W), jnp.float32),
            pltpu.SemaphoreType.DMA,
        ],
    )
    def deg_kernel(dst_hbm, out_hbm, dst_all, ones_v, zero_v, acc_sh, sem_s):
        c = lax.axis_index("c")
        s = lax.axis_index("s")
        wid = c * NS + s

        pltpu.sync_copy(dst_hbm.at[wid], dst_all)

        @pl.loop(0, CH)
        def _(i):
            ones_v[i, :] = jnp.ones((DEG_W,), jnp.float32)

        @pl.loop(0, ZR)
        def _(i):
            zero_v[i, :] = jnp.zeros((DEG_W,), jnp.float32)

        @pl.loop(0, RPS // ZR)
        def _(k):
            pltpu.sync_copy(zero_v, acc_sh.at[pl.ds(s * RPS + k * ZR, ZR)])

        plsc.subcore_barrier()

        GRP = 16
        @pl.loop(0, NCH // GRP)
        def _(g):
            scs = [
                pltpu.async_copy(
                    ones_v, acc_sh.at[dst_all.at[g * GRP + b]], sem_s,
                    add=True)
                for b in range(GRP)
            ]
            for sc_ in scs:
                sc_.wait()

        plsc.subcore_barrier()

        @pl.when(s < NS - 1)
        def _():
            pltpu.sync_copy(
                acc_sh.at[pl.ds(s * RPS, RPS)],
                out_hbm.at[c, pl.ds(s * RPS, RPS)],
            )

        @pl.when(s == NS - 1)
        def _():
            pltpu.sync_copy(
                acc_sh.at[pl.ds((NS - 1) * RPS, N - (NS - 1) * RPS)],
                out_hbm.at[c, pl.ds((NS - 1) * RPS, N - (NS - 1) * RPS)],
            )

    return deg_kernel(dst3)


def _sc_agg(hp, src3, dst3):
    @functools.partial(
        pl.kernel,
        out_type=jax.ShapeDtypeStruct((NC, N, D), jnp.float32),
        mesh=_MESH(),
        scratch_types=[
            pltpu.VMEM((NCH, CH), jnp.int32),
            pltpu.VMEM((NCH, CH), jnp.int32),
            pltpu.VMEM((CH, D), jnp.float32),
            pltpu.VMEM((ZR, D), jnp.float32),
            pltpu.VMEM_SHARED((ACC_ROWS, D), jnp.float32),
            pltpu.SemaphoreType.DMA,
            pltpu.SemaphoreType.DMA,
        ],
    )
    def agg_kernel(hp_hbm, src_hbm, dst_hbm, out_hbm,
                   src_all, dst_all, rows_v, zero_v, acc_sh, sem_g, sem_s):
        c = lax.axis_index("c")
        s = lax.axis_index("s")
        wid = c * NS + s

        pltpu.sync_copy(src_hbm.at[wid], src_all)
        pltpu.sync_copy(dst_hbm.at[wid], dst_all)

        @pl.loop(0, ZR)
        def _(i):
            for j in range(D // 16):
                zero_v[i, pl.ds(j * 16, 16)] = jnp.zeros((16,), jnp.float32)

        @pl.loop(0, RPS // ZR)
        def _(k):
            pltpu.sync_copy(zero_v, acc_sh.at[pl.ds(s * RPS + k * ZR, ZR)])

        plsc.subcore_barrier()

        @pl.loop(0, NCH)
        def _(g):
            pltpu.async_copy(
                hp_hbm.at[src_all.at[g]], rows_v, sem_g).wait()
            pltpu.async_copy(
                rows_v, acc_sh.at[dst_all.at[g]], sem_s, add=True).wait()

        plsc.subcore_barrier()

        @pl.when(s < NS - 1)
        def _():
            pltpu.sync_copy(
                acc_sh.at[pl.ds(s * RPS, RPS)],
                out_hbm.at[c, pl.ds(s * RPS, RPS)],
            )

        @pl.when(s == NS - 1)
        def _():
            pltpu.sync_copy(
                acc_sh.at[pl.ds((NS - 1) * RPS, N - (NS - 1) * RPS)],
                out_hbm.at[c, pl.ds((NS - 1) * RPS, N - (NS - 1) * RPS)],
            )

    return agg_kernel(hp, src3, dst3)


def _tc_matmul(x, W):
    def body(x_ref, w_ref, o_ref):
        o_ref[...] = jnp.dot(x_ref[...], w_ref[...],
                             preferred_element_type=jnp.float32)

    return pl.pallas_call(
        body,
        grid=(N // BLK,),
        in_specs=[
            pl.BlockSpec((BLK, D), lambda i: (i, 0)),
            pl.BlockSpec((D, D), lambda i: (0, 0)),
        ],
        out_specs=pl.BlockSpec((BLK, D), lambda i: (i, 0)),
        out_shape=jax.ShapeDtypeStruct((N, D), jnp.float32),
    )(x, W)


def _tc_prep(deg0, deg1, xW1):
    def body(d0_ref, d1_ref, xw_ref, dmat_ref, hp_ref):
        deg = d0_ref[:, 0:1] + d1_ref[:, 0:1] + 1.0
        dinv = lax.rsqrt(deg)
        dmat = jnp.broadcast_to(dinv, (BLK, D))
        dmat_ref[...] = dmat
        hp_ref[...] = xw_ref[...] * dmat

    return pl.pallas_call(
        body,
        grid=(N // BLK,),
        in_specs=[
            pl.BlockSpec((BLK, DEG_W), lambda i: (i, 0)),
            pl.BlockSpec((BLK, DEG_W), lambda i: (i, 0)),
            pl.BlockSpec((BLK, D), lambda i: (i, 0)),
        ],
        out_specs=[
            pl.BlockSpec((BLK, D), lambda i: (i, 0)),
            pl.BlockSpec((BLK, D), lambda i: (i, 0)),
        ],
        out_shape=[
            jax.ShapeDtypeStruct((N, D), jnp.float32),
            jax.ShapeDtypeStruct((N, D), jnp.float32),
        ],
    )(deg0, deg1, xW1)


def _tc_mid(a, hp, dmat, b2d, W):
    def body(a_ref, hp_ref, dm_ref, b_ref, w_ref, o_ref):
        agg = a_ref[0] + a_ref[1]
        dm = dm_ref[...]
        z = dm * (agg + hp_ref[...]) + b_ref[0:1, :]
        z = jnp.maximum(z, 0.0)
        o_ref[...] = jnp.dot(z, w_ref[...],
                             preferred_element_type=jnp.float32) * dm

    return pl.pallas_call(
        body,
        grid=(N // BLK,),
        in_specs=[
            pl.BlockSpec((NC, BLK, D), lambda i: (0, i, 0)),
            pl.BlockSpec((BLK, D), lambda i: (i, 0)),
            pl.BlockSpec((BLK, D), lambda i: (i, 0)),
            pl.BlockSpec((8, D), lambda i: (0, 0)),
            pl.BlockSpec((D, D), lambda i: (0, 0)),
        ],
        out_specs=pl.BlockSpec((BLK, D), lambda i: (i, 0)),
        out_shape=jax.ShapeDtypeStruct((N, D), jnp.float32),
    )(a, hp, dmat, b2d, W)


def _tc_final(a, hp, dmat, b2d, Wlp, blp):
    def body(a_ref, hp_ref, dm_ref, b_ref, w_ref, bl_ref, o_ref):
        agg = a_ref[0] + a_ref[1]
        z = dm_ref[...] * (agg + hp_ref[...]) + b_ref[0:1, :]
        z = jnp.maximum(z, 0.0)
        y = jnp.dot(z, w_ref[...], preferred_element_type=jnp.float32) \
            + bl_ref[0:1, :]
        o_ref[...] = jax.nn.sigmoid(y)

    return pl.pallas_call(
        body,
        grid=(N // BLK,),
        in_specs=[
            pl.BlockSpec((NC, BLK, D), lambda i: (0, i, 0)),
            pl.BlockSpec((BLK, D), lambda i: (i, 0)),
            pl.BlockSpec((BLK, D), lambda i: (i, 0)),
            pl.BlockSpec((8, D), lambda i: (0, 0)),
            pl.BlockSpec((D, D), lambda i: (0, 0)),
            pl.BlockSpec((8, D), lambda i: (0, 0)),
        ],
        out_specs=pl.BlockSpec((BLK, D), lambda i: (i, 0)),
        out_shape=jax.ShapeDtypeStruct((N, D), jnp.float32),
    )(a, hp, dmat, b2d, Wlp, blp)


def kernel(x, edge_index, W1, b1, W2, b2, Wl, bl):
    ei = edge_index.astype(jnp.int32)
    pad = E_PAD - E
    src = jnp.concatenate([ei[0], jnp.zeros((pad,), jnp.int32)])
    dst = jnp.concatenate(
        [ei[1], N + (jnp.arange(pad, dtype=jnp.int32) % (ACC_ROWS - N))])
    src = src.reshape(NW, NCH, CH)
    dst = dst.reshape(NW, NCH, CH)

    degp = _sc_deg(dst)
    xW1 = _tc_matmul(x, W1)
    dmat, hp1 = _tc_prep(degp[0], degp[1], xW1)

    a1 = _sc_agg(hp1, src, dst)
    b1_2d = jnp.broadcast_to(b1.reshape(1, D), (8, D))
    hp2 = _tc_mid(a1, hp1, dmat, b1_2d, W2)

    a2 = _sc_agg(hp2, src, dst)
    b2_2d = jnp.broadcast_to(b2.reshape(1, D), (8, D))
    Wlp = jnp.pad(Wl, ((0, 0), (0, D - Wl.shape[1])))
    bl_2d = jnp.broadcast_to(bl.reshape(1, 1), (8, D))
    wide = _tc_final(a2, hp2, dmat, b2_2d, Wlp, bl_2d)
    return wide[:, :1]


# speedup vs baseline: 2.6163x; 2.6163x over previous
import dataclasses
import functools

import jax
import jax.numpy as jnp
from jax import lax
from jax.experimental import pallas as pl
from jax.experimental.pallas import tpu as pltpu
from jax.experimental.pallas import tpu_sc as plsc

N = 10000
E = 320000
D = 128

NC = 2
NS = 16
NW = NC * NS
CH = 128
NCH = 80
EPW = NCH * CH
E_PAD = NW * EPW
ACC_ROWS = 10240
RPS = ACC_ROWS // NS
ZR = 64
DEG_W = 16

BLK = 2000


_MESH = functools.partial(
    plsc.VectorSubcoreMesh, core_axis_name="c", subcore_axis_name="s"
)


def _sc_deg(dst3):
    cp = pltpu.CompilerParams()
    if "needs_layout_passes" in pltpu.CompilerParams.__dataclass_fields__:
        cp = dataclasses.replace(cp, needs_layout_passes=False)

    @functools.partial(
        pl.kernel,
        out_type=jax.ShapeDtypeStruct((NW, ACC_ROWS), jnp.float32),
        mesh=_MESH(),
        compiler_params=cp,
        scratch_types=[
            pltpu.VMEM((NCH, CH), jnp.int32),
            pltpu.VMEM((ACC_ROWS,), jnp.float32),
        ],
    )
    def deg_kernel(dst_hbm, out_hbm, dst_all, hist_v):
        c = lax.axis_index("c")
        s = lax.axis_index("s")
        wid = c * NS + s

        pltpu.sync_copy(dst_hbm.at[wid], dst_all)

        @pl.loop(0, ACC_ROWS // 16)
        def _(i):
            hist_v[pl.ds(i * 16, 16)] = jnp.zeros((16,), jnp.float32)

        ones16 = jnp.ones((16,), jnp.float32)

        @pl.loop(0, NCH)
        def _(g):
            for j in range(CH // 16):
                ix = dst_all[g, pl.ds(j * 16, 16)]
                plsc.addupdate_scatter(hist_v, [ix], ones16)

        pltpu.sync_copy(hist_v, out_hbm.at[wid])

    return deg_kernel(dst3)


def _sc_agg(hp, src3, dst3):
    @functools.partial(
        pl.kernel,
        out_type=jax.ShapeDtypeStruct((NC, N, D), jnp.float32),
        mesh=_MESH(),
        scratch_types=[
            pltpu.VMEM((NCH, CH), jnp.int32),
            pltpu.VMEM((NCH, CH), jnp.int32),
            pltpu.VMEM((CH, D), jnp.float32),
            pltpu.VMEM((ZR, D), jnp.float32),
            pltpu.VMEM_SHARED((ACC_ROWS, D), jnp.float32),
            pltpu.SemaphoreType.DMA,
            pltpu.SemaphoreType.DMA,
        ],
    )
    def agg_kernel(hp_hbm, src_hbm, dst_hbm, out_hbm,
                   src_all, dst_all, rows_v, zero_v, acc_sh, sem_g, sem_s):
        c = lax.axis_index("c")
        s = lax.axis_index("s")
        wid = c * NS + s

        pltpu.sync_copy(src_hbm.at[wid], src_all)
        pltpu.sync_copy(dst_hbm.at[wid], dst_all)

        @pl.loop(0, ZR)
        def _(i):
            for j in range(D // 16):
                zero_v[i, pl.ds(j * 16, 16)] = jnp.zeros((16,), jnp.float32)

        @pl.loop(0, RPS // ZR)
        def _(k):
            pltpu.sync_copy(zero_v, acc_sh.at[pl.ds(s * RPS + k * ZR, ZR)])

        plsc.subcore_barrier()

        @pl.loop(0, NCH)
        def _(g):
            pltpu.async_copy(
                hp_hbm.at[src_all.at[g]], rows_v, sem_g).wait()
            pltpu.async_copy(
                rows_v, acc_sh.at[dst_all.at[g]], sem_s, add=True).wait()

        plsc.subcore_barrier()

        @pl.when(s < NS - 1)
        def _():
            pltpu.sync_copy(
                acc_sh.at[pl.ds(s * RPS, RPS)],
                out_hbm.at[c, pl.ds(s * RPS, RPS)],
            )

        @pl.when(s == NS - 1)
        def _():
            pltpu.sync_copy(
                acc_sh.at[pl.ds((NS - 1) * RPS, N - (NS - 1) * RPS)],
                out_hbm.at[c, pl.ds((NS - 1) * RPS, N - (NS - 1) * RPS)],
            )

    return agg_kernel(hp, src3, dst3)


def _tc_matmul(x, W):
    def body(x_ref, w_ref, o_ref):
        o_ref[...] = jnp.dot(x_ref[...], w_ref[...],
                             preferred_element_type=jnp.float32)

    return pl.pallas_call(
        body,
        grid=(N // BLK,),
        in_specs=[
            pl.BlockSpec((BLK, D), lambda i: (i, 0)),
            pl.BlockSpec((D, D), lambda i: (0, 0)),
        ],
        out_specs=pl.BlockSpec((BLK, D), lambda i: (i, 0)),
        out_shape=jax.ShapeDtypeStruct((N, D), jnp.float32),
    )(x, W)


def _tc_prep(dinv16, xW1):
    def body(dv_ref, xw_ref, dmat_ref, hp_ref):
        dmat = jnp.broadcast_to(dv_ref[:, 0:1], (BLK, D))
        dmat_ref[...] = dmat
        hp_ref[...] = xw_ref[...] * dmat

    return pl.pallas_call(
        body,
        grid=(N // BLK,),
        in_specs=[
            pl.BlockSpec((BLK, DEG_W), lambda i: (i, 0)),
            pl.BlockSpec((BLK, D), lambda i: (i, 0)),
        ],
        out_specs=[
            pl.BlockSpec((BLK, D), lambda i: (i, 0)),
            pl.BlockSpec((BLK, D), lambda i: (i, 0)),
        ],
        out_shape=[
            jax.ShapeDtypeStruct((N, D), jnp.float32),
            jax.ShapeDtypeStruct((N, D), jnp.float32),
        ],
    )(dinv16, xW1)


def _tc_mid(a, hp, dmat, b2d, W):
    def body(a_ref, hp_ref, dm_ref, b_ref, w_ref, o_ref):
        agg = a_ref[0] + a_ref[1]
        dm = dm_ref[...]
        z = dm * (agg + hp_ref[...]) + b_ref[0:1, :]
        z = jnp.maximum(z, 0.0)
        o_ref[...] = jnp.dot(z, w_ref[...],
                             preferred_element_type=jnp.float32) * dm

    return pl.pallas_call(
        body,
        grid=(N // BLK,),
        in_specs=[
            pl.BlockSpec((NC, BLK, D), lambda i: (0, i, 0)),
            pl.BlockSpec((BLK, D), lambda i: (i, 0)),
            pl.BlockSpec((BLK, D), lambda i: (i, 0)),
            pl.BlockSpec((8, D), lambda i: (0, 0)),
            pl.BlockSpec((D, D), lambda i: (0, 0)),
        ],
        out_specs=pl.BlockSpec((BLK, D), lambda i: (i, 0)),
        out_shape=jax.ShapeDtypeStruct((N, D), jnp.float32),
    )(a, hp, dmat, b2d, W)


def _tc_final(a, hp, dmat, b2d, Wlp, blp):
    def body(a_ref, hp_ref, dm_ref, b_ref, w_ref, bl_ref, o_ref):
        agg = a_ref[0] + a_ref[1]
        z = dm_ref[...] * (agg + hp_ref[...]) + b_ref[0:1, :]
        z = jnp.maximum(z, 0.0)
        y = jnp.dot(z, w_ref[...], preferred_element_type=jnp.float32) \
            + bl_ref[0:1, :]
        o_ref[...] = jax.nn.sigmoid(y)

    return pl.pallas_call(
        body,
        grid=(N // BLK,),
        in_specs=[
            pl.BlockSpec((NC, BLK, D), lambda i: (0, i, 0)),
            pl.BlockSpec((BLK, D), lambda i: (i, 0)),
            pl.BlockSpec((BLK, D), lambda i: (i, 0)),
            pl.BlockSpec((8, D), lambda i: (0, 0)),
            pl.BlockSpec((D, D), lambda i: (0, 0)),
            pl.BlockSpec((8, D), lambda i: (0, 0)),
        ],
        out_specs=pl.BlockSpec((BLK, D), lambda i: (i, 0)),
        out_shape=jax.ShapeDtypeStruct((N, D), jnp.float32),
    )(a, hp, dmat, b2d, Wlp, blp)


def kernel(x, edge_index, W1, b1, W2, b2, Wl, bl):
    ei = edge_index.astype(jnp.int32)
    pad = E_PAD - E
    src = jnp.concatenate([ei[0], jnp.arange(pad, dtype=jnp.int32) % N])
    dst = jnp.concatenate(
        [ei[1], N + (jnp.arange(pad, dtype=jnp.int32) % (ACC_ROWS - N))])
    src = src.reshape(NW, NCH, CH)
    dst = dst.reshape(NW, NCH, CH)

    degp = _sc_deg(dst)
    xW1 = _tc_matmul(x, W1)
    deg = jnp.sum(degp, axis=0)[:N]
    dinv16 = jnp.broadcast_to(
        lax.rsqrt(deg + 1.0)[:, None], (N, DEG_W))
    dmat, hp1 = _tc_prep(dinv16, xW1)

    a1 = _sc_agg(hp1, src, dst)
    b1_2d = jnp.broadcast_to(b1.reshape(1, D), (8, D))
    hp2 = _tc_mid(a1, hp1, dmat, b1_2d, W2)

    a2 = _sc_agg(hp2, src, dst)
    b2_2d = jnp.broadcast_to(b2.reshape(1, D), (8, D))
    Wlp = jnp.pad(Wl, ((0, 0), (0, D - Wl.shape[1])))
    bl_2d = jnp.broadcast_to(bl.reshape(1, 1), (8, D))
    wide = _tc_final(a2, hp2, dmat, b2_2d, Wlp, bl_2d)
    return wide[:, :1]
